# Initial kernel scaffold; baseline (speedup 1.0000x reference)
import functools

import jax
import jax.numpy as jnp
from jax import lax
from jax.experimental import pallas as pl
from jax.experimental.pallas import tpu as pltpu
from jax.experimental.pallas import tpu_sc as plsc

N_NODES = 10000
D = 128
N_EDGES = 320000

NC = 2
NS = 16
NW = NC * NS

CHUNK = 128
CPW = 80
EPW = CHUNK * CPW
E_PAD = EPW * NW
NPAD = 10240
RPS = NPAD // NS


def _sc_body(x_h, col_h, row_h, ones_h, zagg_h, zdeg_h,
             agg_o, deg_o,
             agg_sh, deg_sh, col_v, row_v, buf0, buf1, ones_v, sem0, sem1):
    cid = lax.axis_index("c")
    sid = lax.axis_index("s")
    wid = cid * NS + sid

    pltpu.sync_copy(col_h.at[wid], col_v)
    pltpu.sync_copy(row_h.at[wid], row_v)
    pltpu.sync_copy(ones_h, ones_v)

    pltpu.sync_copy(zagg_h, agg_sh.at[pl.ds(sid * RPS, RPS)])
    pltpu.sync_copy(zdeg_h, deg_sh.at[pl.ds(sid * RPS, RPS)])
    plsc.subcore_barrier()

    pltpu.async_copy(x_h.at[col_v.at[0]], buf0, sem0).start()
    pltpu.async_copy(x_h.at[col_v.at[1]], buf1, sem1).start()

    def step(i, carry):
        c = i * 2

        pltpu.async_copy(x_h.at[col_v.at[c]], buf0, sem0).wait()
        pltpu.sync_copy(buf0, agg_sh.at[row_v.at[c]], add=True)
        pltpu.sync_copy(ones_v, deg_sh.at[row_v.at[c]], add=True)

        @pl.when(c + 2 < CPW)
        def _():
            pltpu.async_copy(x_h.at[col_v.at[c + 2]], buf0, sem0).start()

        pltpu.async_copy(x_h.at[col_v.at[c + 1]], buf1, sem1).wait()
        pltpu.sync_copy(buf1, agg_sh.at[row_v.at[c + 1]], add=True)
        pltpu.sync_copy(ones_v, deg_sh.at[row_v.at[c + 1]], add=True)

        @pl.when(c + 3 < CPW)
        def _():
            pltpu.async_copy(x_h.at[col_v.at[c + 3]], buf1, sem1).start()

        return carry

    lax.fori_loop(0, CPW // 2, step, 0)

    plsc.subcore_barrier()
    pltpu.sync_copy(agg_sh.at[pl.ds(sid * RPS, RPS)],
                    agg_o.at[cid, pl.ds(sid * RPS, RPS)])
    pltpu.sync_copy(deg_sh.at[pl.ds(sid * RPS, RPS)],
                    deg_o.at[cid, pl.ds(sid * RPS, RPS)])


def _sc_aggregate(x, col3, row3, ones16, zagg, zdeg):
    mesh = plsc.VectorSubcoreMesh(core_axis_name="c", subcore_axis_name="s")
    return pl.kernel(
        _sc_body,
        out_type=[
            jax.ShapeDtypeStruct((NC, NPAD, D), jnp.float32),
            jax.ShapeDtypeStruct((NC, NPAD, 16), jnp.float32),
        ],
        mesh=mesh,
        scratch_types=[
            pltpu.VMEM_SHARED((NPAD, D), jnp.float32),
            pltpu.VMEM_SHARED((NPAD, 16), jnp.float32),
            pltpu.VMEM((CPW, CHUNK), jnp.int32),
            pltpu.VMEM((CPW, CHUNK), jnp.int32),
            pltpu.VMEM((CHUNK, D), jnp.float32),
            pltpu.VMEM((CHUNK, D), jnp.float32),
            pltpu.VMEM((CHUNK, 16), jnp.float32),
            pltpu.SemaphoreType.DMA,
            pltpu.SemaphoreType.DMA,
        ],
    )(x, col3, row3, ones16, zagg, zdeg)


def _tc_body(a_ref, d_ref, wt_ref, b_ref, o_ref):
    agg = a_ref[0] + a_ref[1]
    deg = d_ref[0, :, 0:1] + d_ref[1, :, 0:1]
    deg = jnp.maximum(deg, 1.0)
    h = agg / deg
    acc = jnp.dot(h, wt_ref[...], preferred_element_type=jnp.float32)
    o_ref[...] = jnp.maximum(acc + b_ref[...], 0.0)


def _tc_finish(agg_p, deg_p, wt, b2):
    bm = 1024
    grid = (NPAD // bm,)
    return pl.pallas_call(
        _tc_body,
        grid=grid,
        in_specs=[
            pl.BlockSpec((NC, bm, D), lambda i: (0, i, 0)),
            pl.BlockSpec((NC, bm, 16), lambda i: (0, i, 0)),
            pl.BlockSpec((D, D), lambda i: (0, 0)),
            pl.BlockSpec((1, D), lambda i: (0, 0)),
        ],
        out_specs=pl.BlockSpec((bm, D), lambda i: (i, 0)),
        out_shape=jax.ShapeDtypeStruct((NPAD, D), jnp.float32),
    )(agg_p, deg_p, wt, b2)


def kernel(x, edge_index, W, b):
    row = edge_index[0].astype(jnp.int32)
    col = edge_index[1].astype(jnp.int32)
    pad = E_PAD - N_EDGES
    col_p = jnp.concatenate([col, jnp.zeros((pad,), jnp.int32)])
    row_p = jnp.concatenate([row, jnp.full((pad,), N_NODES, jnp.int32)])
    col3 = col_p.reshape(NW, CPW, CHUNK)
    row3 = row_p.reshape(NW, CPW, CHUNK)

    ones16 = jnp.ones((CHUNK, 16), jnp.float32)
    zagg = jnp.zeros((RPS, D), jnp.float32)
    zdeg = jnp.zeros((RPS, 16), jnp.float32)

    agg_p, deg_p = _sc_aggregate(x, col3, row3, ones16, zagg, zdeg)

    out = _tc_finish(agg_p, deg_p, W.T, b.reshape(1, D))
    return out[:N_NODES]


# speedup vs baseline: 2.3104x; 2.3104x over previous
import jax
import jax.numpy as jnp
from jax import lax
from jax.experimental import pallas as pl
from jax.experimental.pallas import tpu as pltpu
from jax.experimental.pallas import tpu_sc as plsc

N_NODES = 10000
D = 128
N_EDGES = 320000

NC = 2
NS = 16
NW = NC * NS

CHUNK = 64
CPW = 160
EPW = CHUNK * CPW
E_PAD = EPW * NW
NPAD = 10240
RPS = NPAD // NS
KD = RPS // CHUNK


def _sc_body(x_h, col_h, row_h, ones_h, zagg_h, zidx_h,
             agg_o, deg_o,
             agg_sh, deg_sh, cb0, rb0, buf0, ones_v, zidx_v, gsem0):
    cid = lax.axis_index("c")
    sid = lax.axis_index("s")
    wid = cid * NS + sid

    pltpu.sync_copy(ones_h, ones_v)
    pltpu.sync_copy(zidx_h.at[sid], zidx_v)
    pltpu.sync_copy(zagg_h, agg_sh.at[pl.ds(sid * RPS, RPS)])
    for k in range(KD):
        pltpu.sync_copy(ones_v, deg_sh.at[zidx_v.at[k]])
    plsc.subcore_barrier()

    def step(i, carry):
        pltpu.sync_copy(col_h.at[wid, i], cb0)
        pltpu.sync_copy(row_h.at[wid, i], rb0)
        pltpu.async_copy(x_h.at[cb0], buf0, gsem0).wait()
        pltpu.sync_copy(buf0, agg_sh.at[rb0], add=True)
        pltpu.sync_copy(ones_v, deg_sh.at[rb0], add=True)
        return carry

    lax.fori_loop(0, CPW, step, 0)

    plsc.subcore_barrier()
    pltpu.sync_copy(agg_sh.at[pl.ds(sid * RPS, RPS)],
                    agg_o.at[cid, pl.ds(sid * RPS, RPS)])
    for k in range(KD):
        pltpu.sync_copy(deg_sh.at[zidx_v.at[k]], ones_v)
        pltpu.sync_copy(ones_v, deg_o.at[cid, sid, k])


def _sc_aggregate(x, col3, row3, ones16, zagg, zidx):
    mesh = plsc.VectorSubcoreMesh(core_axis_name="c", subcore_axis_name="s")
    return pl.kernel(
        _sc_body,
        out_type=[
            jax.ShapeDtypeStruct((NC, NPAD, D), jnp.float32),
            jax.ShapeDtypeStruct((NC, NS, KD, CHUNK, 16), jnp.float32),
        ],
        mesh=mesh,
        compiler_params=pltpu.CompilerParams(use_tc_tiling_on_sc=False),
        scratch_types=[
            pltpu.VMEM_SHARED((NPAD, D), jnp.float32),
            pltpu.VMEM_SHARED((NPAD, 16), jnp.float32),
            pltpu.VMEM((CHUNK,), jnp.int32),
            pltpu.VMEM((CHUNK,), jnp.int32),
            pltpu.VMEM((CHUNK, D), jnp.float32),
            pltpu.VMEM((CHUNK, 16), jnp.float32),
            pltpu.VMEM((KD, CHUNK), jnp.int32),
            pltpu.SemaphoreType.DMA,
        ],
    )(x, col3, row3, ones16, zagg, zidx)


def _tc_body(a_ref, d_ref, wt_ref, b_ref, o_ref):
    agg = a_ref[0] + a_ref[1]
    deg = d_ref[0, :, 0:1] + d_ref[1, :, 0:1] - 2.0
    deg = jnp.maximum(deg, 1.0)
    h = agg / deg
    acc = jnp.dot(h, wt_ref[...], preferred_element_type=jnp.float32)
    o_ref[...] = jnp.maximum(acc + b_ref[...], 0.0)


def _tc_finish(agg_p, deg_p, wt, b2):
    bm = 1024
    grid = (NPAD // bm,)
    return pl.pallas_call(
        _tc_body,
        grid=grid,
        in_specs=[
            pl.BlockSpec((NC, bm, D), lambda i: (0, i, 0)),
            pl.BlockSpec((NC, bm, 16), lambda i: (0, i, 0)),
            pl.BlockSpec((D, D), lambda i: (0, 0)),
            pl.BlockSpec((1, D), lambda i: (0, 0)),
        ],
        out_specs=pl.BlockSpec((bm, D), lambda i: (i, 0)),
        out_shape=jax.ShapeDtypeStruct((NPAD, D), jnp.float32),
    )(agg_p, deg_p, wt, b2)


def kernel(x, edge_index, W, b):
    row = edge_index[0].astype(jnp.int32)
    col = edge_index[1].astype(jnp.int32)
    pad = E_PAD - N_EDGES
    col_p = jnp.concatenate([col, jnp.zeros((pad,), jnp.int32)])
    row_p = jnp.concatenate([row, jnp.full((pad,), N_NODES, jnp.int32)])
    col3 = col_p.reshape(NW, CPW, CHUNK)
    row3 = row_p.reshape(NW, CPW, CHUNK)

    ones16 = jnp.ones((CHUNK, 16), jnp.float32)
    zagg = jnp.zeros((RPS, D), jnp.float32)
    zidx = (jnp.arange(NS, dtype=jnp.int32)[:, None, None] * RPS
            + jnp.arange(KD, dtype=jnp.int32)[None, :, None] * CHUNK
            + jnp.arange(CHUNK, dtype=jnp.int32)[None, None, :])

    agg_p, deg_p = _sc_aggregate(x, col3, row3, ones16, zagg, zidx)

    deg_p = deg_p.reshape(NC, NPAD, 16)
    out = _tc_finish(agg_p, deg_p, W.T, b.reshape(1, D))
    return out[:N_NODES]


# speedup vs baseline: 2.5991x; 1.1250x over previous
import jax
import jax.numpy as jnp
from jax import lax
from jax.experimental import pallas as pl
from jax.experimental.pallas import tpu as pltpu
from jax.experimental.pallas import tpu_sc as plsc

N_NODES = 10000
D = 128
N_EDGES = 320000

NC = 2
NS = 16
NW = NC * NS

CHUNK = 64
CPW = 160
EPW = CHUNK * CPW
E_PAD = EPW * NW
NPAD = 10240
RPS = NPAD // NS
KD = RPS // CHUNK


def _sc_body(x_h, col_h, row_h, ones_h, zagg_h, zidx_h,
             agg_o, deg_o,
             agg_sh, deg_sh, col_v, row_v, buf0, buf1, ones_v, zidx_v,
             gsem0, gsem1):
    cid = lax.axis_index("c")
    sid = lax.axis_index("s")
    wid = cid * NS + sid

    pltpu.sync_copy(ones_h, ones_v)
    pltpu.sync_copy(zidx_h.at[sid], zidx_v)
    pltpu.sync_copy(col_h.at[wid], col_v)
    pltpu.sync_copy(row_h.at[wid], row_v)
    pltpu.sync_copy(zagg_h, agg_sh.at[pl.ds(sid * RPS, RPS)])
    for k in range(KD):
        pltpu.sync_copy(ones_v, deg_sh.at[zidx_v.at[k]])
    plsc.subcore_barrier()

    pltpu.async_copy(x_h.at[col_v.at[0]], buf0, gsem0)
    pltpu.async_copy(x_h.at[col_v.at[1]], buf1, gsem1)

    def step(i, carry):
        c = i * 2

        pltpu.make_async_copy(x_h.at[col_v.at[c]], buf0, gsem0).wait()
        pltpu.sync_copy(buf0, agg_sh.at[row_v.at[c]], add=True)
        pltpu.sync_copy(ones_v, deg_sh.at[row_v.at[c]], add=True)
        pltpu.async_copy(x_h.at[col_v.at[c + 2]], buf0, gsem0)

        pltpu.make_async_copy(x_h.at[col_v.at[c + 1]], buf1, gsem1).wait()
        pltpu.sync_copy(buf1, agg_sh.at[row_v.at[c + 1]], add=True)
        pltpu.sync_copy(ones_v, deg_sh.at[row_v.at[c + 1]], add=True)
        pltpu.async_copy(x_h.at[col_v.at[c + 3]], buf1, gsem1)

        return carry

    lax.fori_loop(0, CPW // 2, step, 0)
    pltpu.make_async_copy(x_h.at[col_v.at[CPW]], buf0, gsem0).wait()
    pltpu.make_async_copy(x_h.at[col_v.at[CPW + 1]], buf1, gsem1).wait()

    plsc.subcore_barrier()
    pltpu.sync_copy(agg_sh.at[pl.ds(sid * RPS, RPS)],
                    agg_o.at[cid, pl.ds(sid * RPS, RPS)])
    for k in range(KD):
        pltpu.sync_copy(deg_sh.at[zidx_v.at[k]], ones_v)
        pltpu.sync_copy(ones_v, deg_o.at[cid, sid, k])


def _sc_aggregate(x, col3, row3, ones16, zagg, zidx):
    mesh = plsc.VectorSubcoreMesh(core_axis_name="c", subcore_axis_name="s")
    return pl.kernel(
        _sc_body,
        out_type=[
            jax.ShapeDtypeStruct((NC, NPAD, D), jnp.float32),
            jax.ShapeDtypeStruct((NC, NS, KD, CHUNK, 16), jnp.float32),
        ],
        mesh=mesh,
        compiler_params=pltpu.CompilerParams(use_tc_tiling_on_sc=False),
        scratch_types=[
            pltpu.VMEM_SHARED((NPAD, D), jnp.float32),
            pltpu.VMEM_SHARED((NPAD, 16), jnp.float32),
            pltpu.VMEM((CPW + 2, CHUNK), jnp.int32),
            pltpu.VMEM((CPW, CHUNK), jnp.int32),
            pltpu.VMEM((CHUNK, D), jnp.float32),
            pltpu.VMEM((CHUNK, D), jnp.float32),
            pltpu.VMEM((CHUNK, 16), jnp.float32),
            pltpu.VMEM((KD, CHUNK), jnp.int32),
            pltpu.SemaphoreType.DMA,
            pltpu.SemaphoreType.DMA,
        ],
    )(x, col3, row3, ones16, zagg, zidx)


def _tc_body(a_ref, d_ref, wt_ref, b_ref, o_ref):
    agg = a_ref[0] + a_ref[1]
    deg = d_ref[0, :, 0:1] + d_ref[1, :, 0:1] - 2.0
    deg = jnp.maximum(deg, 1.0)
    h = agg / deg
    acc = jnp.dot(h, wt_ref[...], preferred_element_type=jnp.float32)
    o_ref[...] = jnp.maximum(acc + b_ref[...], 0.0)


def _tc_finish(agg_p, deg_p, wt, b2):
    bm = 1024
    grid = (NPAD // bm,)
    return pl.pallas_call(
        _tc_body,
        grid=grid,
        in_specs=[
            pl.BlockSpec((NC, bm, D), lambda i: (0, i, 0)),
            pl.BlockSpec((NC, bm, 16), lambda i: (0, i, 0)),
            pl.BlockSpec((D, D), lambda i: (0, 0)),
            pl.BlockSpec((1, D), lambda i: (0, 0)),
        ],
        out_specs=pl.BlockSpec((bm, D), lambda i: (i, 0)),
        out_shape=jax.ShapeDtypeStruct((NPAD, D), jnp.float32),
    )(agg_p, deg_p, wt, b2)


def kernel(x, edge_index, W, b):
    row = edge_index[0].astype(jnp.int32)
    col = edge_index[1].astype(jnp.int32)
    pad = E_PAD - N_EDGES
    col_p = jnp.concatenate([col, jnp.zeros((pad,), jnp.int32)])
    row_p = jnp.concatenate([row, jnp.full((pad,), N_NODES, jnp.int32)])
    col3 = jnp.concatenate(
        [col_p.reshape(NW, CPW, CHUNK),
         jnp.zeros((NW, 2, CHUNK), jnp.int32)], axis=1)
    row3 = row_p.reshape(NW, CPW, CHUNK)

    ones16 = jnp.ones((CHUNK, 16), jnp.float32)
    zagg = jnp.zeros((RPS, D), jnp.float32)
    zidx = (jnp.arange(NS, dtype=jnp.int32)[:, None, None] * RPS
            + jnp.arange(KD, dtype=jnp.int32)[None, :, None] * CHUNK
            + jnp.arange(CHUNK, dtype=jnp.int32)[None, None, :])

    agg_p, deg_p = _sc_aggregate(x, col3, row3, ones16, zagg, zidx)

    deg_p = deg_p.reshape(NC, NPAD, 16)
    out = _tc_finish(agg_p, deg_p, W.T, b.reshape(1, D))
    return out[:N_NODES]
